# baseline (speedup 1.0000x reference)
import functools

import jax
import jax.numpy as jnp
from jax import lax
from jax.experimental import pallas as pl
from jax.experimental.pallas import tpu as pltpu
from jax.experimental.pallas import tpu_sc as plsc

CHUNK = 128
LANE = 16
PITCH = 129


def _transpose_table(weight_t, tail_flat, V, D, NC, NS, mesh):
    NW = NC * NS
    full_cols = V // CHUNK
    tail = V - full_cols * CHUNK
    base, extra = divmod(full_cols, NW)

    @functools.partial(
        pl.kernel,
        out_type=jax.ShapeDtypeStruct((V * D,), jnp.float32),
        mesh=mesh,
        scratch_types=[
            pltpu.VMEM((D, PITCH), jnp.float32),
            pltpu.VMEM((CHUNK * D,), jnp.float32),
            pltpu.SemaphoreType.DMA,
        ],
        compiler_params=pltpu.CompilerParams(
            use_tc_tiling_on_sc=True, needs_layout_passes=False
        ),
    )
    def detile(wt_hbm, tail_hbm, out_hbm, in_v, tr_v, sem):
        wid = lax.axis_index("s") * NC + lax.axis_index("c")
        j0 = wid * base + lax.min(wid, extra)
        j1 = j0 + base + jnp.where(wid < extra, 1, 0)
        rows0 = lax.iota(jnp.int32, LANE)
        rows1 = rows0 + LANE

        def do_cols(j, width):
            for i in range(D // 8):
                pltpu.async_copy(
                    wt_hbm.at[pl.ds(8 * i, 8), pl.ds(j * CHUNK, width)],
                    in_v.at[pl.ds(8 * i, 8), pl.ds(0, width)],
                    sem,
                ).wait()

            @pl.loop(0, width)
            def _(b):
                bvec = jnp.zeros((LANE,), jnp.int32) + b
                v0 = plsc.load_gather(in_v, [rows0, bvec])
                v1 = plsc.load_gather(in_v, [rows1, bvec])
                tr_v[pl.ds(b * D, LANE)] = v0
                tr_v[pl.ds(b * D + LANE, LANE)] = v1

            pltpu.sync_copy(
                tr_v.at[pl.ds(0, width * D)],
                out_hbm.at[pl.ds(j * CHUNK * D, width * D)],
            )

        @pl.loop(j0, j1)
        def _(j):
            do_cols(j, CHUNK)

        if tail:
            @pl.when(wid == NW - 1)
            def _():
                pltpu.sync_copy(tail_hbm, tr_v.at[pl.ds(0, tail * D)])
                pltpu.sync_copy(
                    tr_v.at[pl.ds(0, tail * D)],
                    out_hbm.at[pl.ds(full_cols * CHUNK * D, tail * D)],
                )

    return detile(weight_t, tail_flat)


def _gather_rows(idx, table, N, D, NC, NS, mesh):
    NW = NC * NS
    G, K = idx.shape[1], idx.shape[2]
    GROUP = K * CHUNK
    table2d = table.reshape(-1, D)

    @functools.partial(
        pl.kernel,
        out_type=jax.ShapeDtypeStruct((NW, G, GROUP, D), jnp.float32),
        mesh=mesh,
        scratch_types=[
            pltpu.VMEM((G, K, CHUNK), jnp.int32),
            pltpu.VMEM((2, GROUP, D), jnp.float32),
            pltpu.SemaphoreType.DMA,
            pltpu.SemaphoreType.DMA,
            pltpu.SemaphoreType.DMA,
        ],
        compiler_params=pltpu.CompilerParams(use_tc_tiling_on_sc=False),
    )
    def emb(idx_hbm, table_hbm, out_hbm, idx_v, rows_v, gsem, ssem0, ssem1):
        wid = lax.axis_index("s") * NC + lax.axis_index("c")
        pltpu.sync_copy(idx_hbm.at[wid], idx_v)
        ssems = (ssem0, ssem1)

        def gather_group(g, b):
            descs = [
                pltpu.async_copy(
                    table_hbm.at[idx_v.at[g, k]],
                    rows_v.at[b, pl.ds(k * CHUNK, CHUNK)],
                    gsem,
                )
                for k in range(K)
            ]
            for d_ in descs:
                d_.wait()

        def fire_store(g, b):
            pltpu.async_copy(rows_v.at[b], out_hbm.at[wid, g], ssems[b])

        def wait_store(b):
            pltpu.make_async_copy(rows_v.at[b], out_hbm.at[wid, 0], ssems[b]).wait()

        gather_group(0, 0)
        fire_store(0, 0)
        gather_group(1, 1)
        fire_store(1, 1)

        @pl.loop(2, G, step=2)
        def _(g):
            for b in range(2):
                wait_store(b)
                gather_group(g + b, b)
                fire_store(g + b, b)

        wait_store(0)
        wait_store(1)

    return emb(idx, table2d)


def kernel(indices, weight):
    B, F = indices.shape
    V, D = weight.shape
    N = B * F

    info = plsc.get_sparse_core_info()
    NC, NS = info.num_cores, info.num_subcores
    NW = NC * NS
    per_w = N // NW
    n_chunks = per_w // CHUNK
    K = 13
    G = n_chunks // K
    assert per_w * NW == N and n_chunks * CHUNK == per_w
    assert G * K == n_chunks and G % 2 == 0

    mesh = plsc.VectorSubcoreMesh(core_axis_name="c", subcore_axis_name="s")

    n_tail = V % CHUNK
    tail_flat = lax.slice(weight, (V - n_tail, 0), (V, D)).reshape(n_tail * D)
    flat_table = _transpose_table(weight.T, tail_flat, V, D, NC, NS, mesh)
    idx = indices.reshape(NW, G, K, CHUNK).astype(jnp.int32)
    out = _gather_rows(idx, flat_table, N, D, NC, NS, mesh)
    return out.reshape(B, F, D)


# speedup vs baseline: 1.5779x; 1.5779x over previous
import functools

import jax
import jax.numpy as jnp
from jax import lax
from jax.experimental import pallas as pl
from jax.experimental.pallas import tpu as pltpu
from jax.experimental.pallas import tpu_sc as plsc

CHUNK = 128
LANE = 16
PITCH = 129


def _transpose_table(weight_t, tail_flat, V, D, NC, NS, mesh):
    NW = NC * NS
    full_cols = V // CHUNK
    tail = V - full_cols * CHUNK
    base, extra = divmod(full_cols, NW)

    assert base % 2 == 0

    @functools.partial(
        pl.kernel,
        out_type=jax.ShapeDtypeStruct((V * D,), jnp.float32),
        mesh=mesh,
        scratch_types=[
            pltpu.VMEM((2, D, PITCH), jnp.float32),
            pltpu.VMEM((2, CHUNK * D), jnp.float32),
            pltpu.SemaphoreType.DMA,
            pltpu.SemaphoreType.DMA,
            pltpu.SemaphoreType.DMA,
            pltpu.SemaphoreType.DMA,
        ],
        compiler_params=pltpu.CompilerParams(
            use_tc_tiling_on_sc=True, needs_layout_passes=False
        ),
    )
    def detile(wt_hbm, tail_hbm, out_hbm, in_v, tr_v, is0, is1, os0, os1):
        wid = lax.axis_index("s") * NC + lax.axis_index("c")
        j0 = wid * base + lax.min(wid, extra)
        rows0 = lax.iota(jnp.int32, LANE)
        rows1 = rows0 + LANE
        isems = (is0, is1)
        osems = (os0, os1)

        def fire_in(j, b):
            jc = lax.min(j, full_cols - 1)
            for i in range(D // 8):
                pltpu.async_copy(
                    wt_hbm.at[pl.ds(8 * i, 8), pl.ds(jc * CHUNK, CHUNK)],
                    in_v.at[b, pl.ds(8 * i, 8), pl.ds(0, CHUNK)],
                    isems[b],
                )

        def wait_in(b):
            pltpu.make_async_copy(
                wt_hbm.at[pl.ds(0, D), pl.ds(0, CHUNK)],
                in_v.at[b, pl.ds(0, D), pl.ds(0, CHUNK)],
                isems[b],
            ).wait()

        def compute(b):
            @pl.loop(0, CHUNK, unroll=8)
            def _(c):
                bvec = jnp.zeros((LANE,), jnp.int32) + c
                v0 = plsc.load_gather(in_v.at[b], [rows0, bvec])
                v1 = plsc.load_gather(in_v.at[b], [rows1, bvec])
                tr_v[b, pl.ds(c * D, LANE)] = v0
                tr_v[b, pl.ds(c * D + LANE, LANE)] = v1

        def fire_store(j, b):
            pltpu.async_copy(
                tr_v.at[b], out_hbm.at[pl.ds(j * CHUNK * D, CHUNK * D)], osems[b]
            )

        def wait_store(b):
            pltpu.make_async_copy(
                tr_v.at[b], out_hbm.at[pl.ds(0, CHUNK * D)], osems[b]
            ).wait()

        fire_in(j0, 0)
        fire_in(j0 + 1, 1)
        for b in range(2):
            wait_in(b)
            compute(b)
            fire_store(j0 + b, b)
            fire_in(j0 + 2 + b, b)

        @pl.loop(1, base // 2)
        def _(p):
            j = j0 + 2 * p
            for b in range(2):
                wait_in(b)
                wait_store(b)
                compute(b)
                fire_store(j + b, b)
                fire_in(j + 2 + b, b)

        for b in range(2):
            wait_in(b)
            wait_store(b)

        if extra:
            @pl.when(wid < extra)
            def _():
                je = j0 + base
                fire_in(je, 0)
                wait_in(0)
                compute(0)
                fire_store(je, 0)
                wait_store(0)

        if tail:
            @pl.when(wid == NW - 1)
            def _():
                pltpu.sync_copy(tail_hbm, tr_v.at[0, pl.ds(0, tail * D)])
                pltpu.sync_copy(
                    tr_v.at[0, pl.ds(0, tail * D)],
                    out_hbm.at[pl.ds(full_cols * CHUNK * D, tail * D)],
                )

    return detile(weight_t, tail_flat)


def _gather_rows(idx, table, N, D, NC, NS, mesh):
    NW = NC * NS
    G, K = idx.shape[1], idx.shape[2]
    GROUP = K * CHUNK
    table2d = table.reshape(-1, D)

    @functools.partial(
        pl.kernel,
        out_type=jax.ShapeDtypeStruct((NW, G, GROUP, D), jnp.float32),
        mesh=mesh,
        scratch_types=[
            pltpu.VMEM((G, K, CHUNK), jnp.int32),
            pltpu.VMEM((2, GROUP, D), jnp.float32),
            pltpu.SemaphoreType.DMA,
            pltpu.SemaphoreType.DMA,
            pltpu.SemaphoreType.DMA,
        ],
        compiler_params=pltpu.CompilerParams(use_tc_tiling_on_sc=False),
    )
    def emb(idx_hbm, table_hbm, out_hbm, idx_v, rows_v, gsem, ssem0, ssem1):
        wid = lax.axis_index("s") * NC + lax.axis_index("c")
        pltpu.sync_copy(idx_hbm.at[wid], idx_v)
        ssems = (ssem0, ssem1)

        def gather_group(g, b):
            descs = [
                pltpu.async_copy(
                    table_hbm.at[idx_v.at[g, k]],
                    rows_v.at[b, pl.ds(k * CHUNK, CHUNK)],
                    gsem,
                )
                for k in range(K)
            ]
            for d_ in descs:
                d_.wait()

        def fire_store(g, b):
            pltpu.async_copy(rows_v.at[b], out_hbm.at[wid, g], ssems[b])

        def wait_store(b):
            pltpu.make_async_copy(rows_v.at[b], out_hbm.at[wid, 0], ssems[b]).wait()

        gather_group(0, 0)
        fire_store(0, 0)
        gather_group(1, 1)
        fire_store(1, 1)

        @pl.loop(2, G, step=2)
        def _(g):
            for b in range(2):
                wait_store(b)
                gather_group(g + b, b)
                fire_store(g + b, b)

        wait_store(0)
        wait_store(1)

    return emb(idx, table2d)


def kernel(indices, weight):
    B, F = indices.shape
    V, D = weight.shape
    N = B * F

    info = plsc.get_sparse_core_info()
    NC, NS = info.num_cores, info.num_subcores
    NW = NC * NS
    per_w = N // NW
    n_chunks = per_w // CHUNK
    K = 13
    G = n_chunks // K
    assert per_w * NW == N and n_chunks * CHUNK == per_w
    assert G * K == n_chunks and G % 2 == 0

    mesh = plsc.VectorSubcoreMesh(core_axis_name="c", subcore_axis_name="s")

    n_tail = V % CHUNK
    tail_flat = lax.slice(weight, (V - n_tail, 0), (V, D)).reshape(n_tail * D)
    flat_table = _transpose_table(weight.T, tail_flat, V, D, NC, NS, mesh)
    idx = indices.reshape(NW, G, K, CHUNK).astype(jnp.int32)
    out = _gather_rows(idx, flat_table, N, D, NC, NS, mesh)
    return out.reshape(B, F, D)
